# Initial kernel scaffold; baseline (speedup 1.0000x reference)
import jax
import jax.numpy as jnp
from jax import lax
from jax.experimental import pallas as pl
from jax.experimental.pallas import tpu as pltpu
from jax.experimental.pallas import tpu_sc as plsc

C = 160
NPTS = 500
NP_NEIGH = 8
L = 16
NPAD = 512
NC = 2
NS = 16
NW = NC * NS
CPW = C // NW
NBLK = NPAD // L


def _sc_body(f1_hbm, f2_hbm, inds_hbm, out_hbm, f1_v, f2_v, inds_v, out_v):
    wid = lax.axis_index("s") * NC + lax.axis_index("c")
    c0 = wid * CPW
    pltpu.sync_copy(f1_hbm.at[pl.ds(c0, CPW)], f1_v)
    pltpu.sync_copy(f2_hbm.at[pl.ds(c0, CPW)], f2_v)
    pltpu.sync_copy(inds_hbm, inds_v)

    row_ids = [jnp.full((L,), c, jnp.int32) for c in range(CPW)]

    def block(blk, carry):
        base = blk * L
        idxs = [inds_v[n, pl.ds(base, L)] for n in range(NP_NEIGH)]
        for c in range(CPW):
            acc = plsc.load_gather(f2_v, [row_ids[c], idxs[0]])
            for n in range(1, NP_NEIGH):
                acc = acc + plsc.load_gather(f2_v, [row_ids[c], idxs[n]])
            out_v[c, pl.ds(base, L)] = acc * f1_v[c, pl.ds(base, L)] * 0.125
        return carry

    lax.fori_loop(0, NBLK, block, 0)
    pltpu.sync_copy(out_v, out_hbm.at[pl.ds(c0, CPW)])


@jax.jit
def kernel(feat1, feat2, inds):
    f1 = jnp.pad(feat1[0], ((0, 0), (0, NPAD - NPTS)))
    f2 = jnp.pad(feat2[0], ((0, 0), (0, NPAD - NPTS)))
    it = jnp.pad(inds[0].astype(jnp.int32).T, ((0, 0), (0, NPAD - NPTS)))

    mesh = plsc.VectorSubcoreMesh(core_axis_name="c", subcore_axis_name="s")
    out = pl.kernel(
        _sc_body,
        out_type=jax.ShapeDtypeStruct((C, NPAD), jnp.float32),
        mesh=mesh,
        scratch_types=[
            pltpu.VMEM((CPW, NPAD), jnp.float32),
            pltpu.VMEM((CPW, NPAD), jnp.float32),
            pltpu.VMEM((NP_NEIGH, NPAD), jnp.int32),
            pltpu.VMEM((CPW, NPAD), jnp.float32),
        ],
    )(f1, f2, it)
    return out[None, :, :NPTS]


# speedup vs baseline: 224.7925x; 224.7925x over previous
import jax
import jax.numpy as jnp
from jax import lax
from jax.experimental import pallas as pl
from jax.experimental.pallas import tpu as pltpu
from jax.experimental.pallas import tpu_sc as plsc

C = 160
NPTS = 500
NP_NEIGH = 8
L = 16
NPAD = 512
NC = 2
NS = 16
NW = NC * NS
CPW = C // NW
NBLK = NPAD // L


def _sc_body(f1_hbm, f2_hbm, inds_hbm, out_hbm, f1_v, f2_v, inds_v, out_v):
    wid = lax.axis_index("s") * NC + lax.axis_index("c")
    e0 = wid * (CPW * NPAD)
    pltpu.sync_copy(f1_hbm.at[pl.ds(e0, CPW * NPAD)], f1_v)
    pltpu.sync_copy(f2_hbm.at[pl.ds(e0, CPW * NPAD)], f2_v)
    pltpu.sync_copy(inds_hbm, inds_v)

    row_off = [jnp.full((L,), c * NPAD, jnp.int32) for c in range(CPW)]

    def block(blk, carry):
        base = blk * L
        idxs = [inds_v[pl.ds(n * NPAD + base, L)] for n in range(NP_NEIGH)]
        for c in range(CPW):
            acc = plsc.load_gather(f2_v, [row_off[c] + idxs[0]])
            for n in range(1, NP_NEIGH):
                acc = acc + plsc.load_gather(f2_v, [row_off[c] + idxs[n]])
            ds = pl.ds(c * NPAD + base, L)
            out_v[ds] = acc * f1_v[ds] * 0.125
        return carry

    lax.fori_loop(0, NBLK, block, 0)
    pltpu.sync_copy(out_v, out_hbm.at[pl.ds(e0, CPW * NPAD)])


@jax.jit
def kernel(feat1, feat2, inds):
    f1 = jnp.pad(feat1[0], ((0, 0), (0, NPAD - NPTS))).reshape(-1)
    f2 = jnp.pad(feat2[0], ((0, 0), (0, NPAD - NPTS))).reshape(-1)
    it = jnp.pad(inds[0].astype(jnp.int32).T,
                 ((0, 0), (0, NPAD - NPTS))).reshape(-1)

    mesh = plsc.VectorSubcoreMesh(core_axis_name="c", subcore_axis_name="s")
    out = pl.kernel(
        _sc_body,
        out_type=jax.ShapeDtypeStruct((C * NPAD,), jnp.float32),
        mesh=mesh,
        compiler_params=pltpu.CompilerParams(needs_layout_passes=False),
        scratch_types=[
            pltpu.VMEM((CPW * NPAD,), jnp.float32),
            pltpu.VMEM((CPW * NPAD,), jnp.float32),
            pltpu.VMEM((NP_NEIGH * NPAD,), jnp.int32),
            pltpu.VMEM((CPW * NPAD,), jnp.float32),
        ],
    )(f1, f2, it)
    return out.reshape(1, C, NPAD)[:, :, :NPTS]
